# baseline (speedup 1.0000x reference)
import functools

import jax
import jax.numpy as jnp
from jax import lax
from jax.experimental import pallas as pl
from jax.experimental.pallas import tpu as pltpu
from jax.experimental.pallas import tpu_sc as plsc

NUM_FIELDS = 26
VOCAB = 100
VOCAB_PAD = 128
EMB_DIM = 64
BATCH = 16384

NC = 2
NS = 16
NW = NC * NS
LANES = 16

BBLK = 128
NBLK = NUM_FIELDS * (BATCH // BBLK)
BPW = NBLK // NW
RPW = BPW * BBLK
TDS = EMB_DIM // 8

_mesh = plsc.VectorSubcoreMesh(
    core_axis_name="c", subcore_axis_name="s", num_cores=NC, num_subcores=NS
)


@functools.partial(
    pl.kernel,
    out_type=jax.ShapeDtypeStruct((NUM_FIELDS, EMB_DIM, BATCH), jnp.float32),
    mesh=_mesh,
    scratch_types=[
        pltpu.VMEM((RPW,), jnp.int32),
        pltpu.VMEM((2 * EMB_DIM * VOCAB_PAD,), jnp.float32),
        pltpu.VMEM((EMB_DIM, BBLK), jnp.float32),
        pltpu.VMEM((EMB_DIM, BBLK), jnp.float32),
        pltpu.SemaphoreType.DMA,
        pltpu.SemaphoreType.DMA,
    ],
    compiler_params=pltpu.CompilerParams(
        use_tc_tiling_on_sc=True, needs_layout_passes=False
    ),
)
def _emb_lookup(
    xt_hbm, tbl_hbm, out_hbm, raw_v, tbl_v, buf0_v, buf1_v, sem0, sem1,
):
    wid = lax.axis_index("s") * NC + lax.axis_index("c")
    g0 = wid * BPW
    f0 = g0 // (BATCH // BBLK)
    f1 = jnp.minimum(f0 + 1, NUM_FIELDS - 1)

    FSLICE = EMB_DIM * VOCAB_PAD
    stage = [
        pltpu.async_copy(xt_hbm.at[pl.ds(g0 * BBLK, RPW)], raw_v, sem0),
        pltpu.async_copy(
            tbl_hbm.at[pl.ds(f0 * FSLICE, FSLICE)], tbl_v.at[pl.ds(0, FSLICE)], sem1
        ),
        pltpu.async_copy(
            tbl_hbm.at[pl.ds(f1 * FSLICE, FSLICE)],
            tbl_v.at[pl.ds(FSLICE, FSLICE)],
            sem1,
        ),
    ]
    for cp in stage:
        cp.wait()

    bufs = (buf0_v, buf1_v)
    sems = (sem0, sem1)
    NBUF = len(bufs)

    def build_block(i, buf):
        g = g0 + i
        sel = g // (BATCH // BBLK) - f0
        tbase = sel * FSLICE
        idx = [raw_v[pl.ds(i * BBLK + bg * LANES, LANES)] for bg in range(8)]

        @plsc.parallel_loop(0, EMB_DIM, step=1, unroll=16)
        def per_d4(dd):
            for u in range(1):
                d = dd + u
                base16 = jnp.full((LANES,), tbase + d * VOCAB_PAD, jnp.int32)
                for bg in range(8):
                    buf[d, pl.ds(bg * LANES, LANES)] = plsc.load_gather(
                        tbl_v, [base16 + idx[bg]]
                    )

    def fire(i, buf, sem):
        g = g0 + i
        fld = g // (BATCH // BBLK)
        b0 = (g % (BATCH // BBLK)) * BBLK
        pltpu.async_copy(
            buf, out_hbm.at[fld, pl.ds(0, EMB_DIM), pl.ds(b0, BBLK)], sem
        )

    def drain(buf, sem):
        pltpu.make_async_copy(
            out_hbm.at[0, pl.ds(0, EMB_DIM), pl.ds(0, BBLK)], buf, sem
        ).wait()

    def ring(j, carry):
        for half in range(NBUF):
            i = j * NBUF + half

            @pl.when(j > 0)
            def _():
                drain(bufs[half], sems[half])

            build_block(i, bufs[half])
            fire(i, bufs[half], sems[half])
        return carry

    lax.fori_loop(0, BPW // NBUF, ring, 0)
    for half in range(NBUF):
        drain(bufs[half], sems[half])


def kernel(x_cat, tables):
    xt_flat = x_cat.astype(jnp.int32).T.reshape(-1)
    tbl = jnp.pad(
        tables.transpose(0, 2, 1), ((0, 0), (0, 0), (0, VOCAB_PAD - VOCAB))
    ).reshape(-1)
    out = _emb_lookup(xt_flat, tbl)
    return jnp.transpose(out, (2, 0, 1))


# speedup vs baseline: 1.0893x; 1.0893x over previous
import functools

import jax
import jax.numpy as jnp
from jax import lax
from jax.experimental import pallas as pl
from jax.experimental.pallas import tpu as pltpu
from jax.experimental.pallas import tpu_sc as plsc

NUM_FIELDS = 26
VOCAB = 100
VOCAB_PAD = 128
EMB_DIM = 64
BATCH = 16384

NC = 2
NS = 16
NW = NC * NS
LANES = 16

BBLK = 256
NBLK = NUM_FIELDS * (BATCH // BBLK)
BPW = NBLK // NW
RPW = BPW * BBLK
TDS = EMB_DIM // 8

_mesh = plsc.VectorSubcoreMesh(
    core_axis_name="c", subcore_axis_name="s", num_cores=NC, num_subcores=NS
)


@functools.partial(
    pl.kernel,
    out_type=jax.ShapeDtypeStruct((NUM_FIELDS, EMB_DIM, BATCH), jnp.float32),
    mesh=_mesh,
    scratch_types=[
        pltpu.VMEM((RPW,), jnp.int32),
        pltpu.VMEM((2 * EMB_DIM * VOCAB_PAD,), jnp.float32),
        pltpu.VMEM((EMB_DIM, BBLK), jnp.float32),
        pltpu.VMEM((EMB_DIM, BBLK), jnp.float32),
        pltpu.SemaphoreType.DMA,
        pltpu.SemaphoreType.DMA,
    ],
    compiler_params=pltpu.CompilerParams(
        use_tc_tiling_on_sc=True, needs_layout_passes=False
    ),
)
def _emb_lookup(
    xt_hbm, tbl_hbm, out_hbm, raw_v, tbl_v, buf0_v, buf1_v, sem0, sem1,
):
    wid = lax.axis_index("s") * NC + lax.axis_index("c")
    g0 = wid * BPW
    f0 = g0 // (BATCH // BBLK)
    f1 = jnp.minimum(f0 + 1, NUM_FIELDS - 1)

    FSLICE = EMB_DIM * VOCAB_PAD
    stage = [
        pltpu.async_copy(xt_hbm.at[pl.ds(g0 * BBLK, RPW)], raw_v, sem0),
        pltpu.async_copy(
            tbl_hbm.at[pl.ds(f0 * FSLICE, FSLICE)], tbl_v.at[pl.ds(0, FSLICE)], sem1
        ),
        pltpu.async_copy(
            tbl_hbm.at[pl.ds(f1 * FSLICE, FSLICE)],
            tbl_v.at[pl.ds(FSLICE, FSLICE)],
            sem1,
        ),
    ]
    for cp in stage:
        cp.wait()

    bufs = (buf0_v, buf1_v)
    sems = (sem0, sem1)
    NBUF = len(bufs)

    def build_block(i, buf):
        g = g0 + i
        sel = g // (BATCH // BBLK) - f0
        tbase = sel * FSLICE
        idx = [raw_v[pl.ds(i * BBLK + bg * LANES, LANES)] for bg in range(BBLK // LANES)]

        @plsc.parallel_loop(0, EMB_DIM, step=1, unroll=4)
        def per_d4(dd):
            for u in range(1):
                d = dd + u
                base16 = jnp.full((LANES,), tbase + d * VOCAB_PAD, jnp.int32)
                for bg in range(BBLK // LANES):
                    buf[d, pl.ds(bg * LANES, LANES)] = plsc.load_gather(
                        tbl_v, [base16 + idx[bg]]
                    )

    def fire(i, buf, sem):
        g = g0 + i
        fld = g // (BATCH // BBLK)
        b0 = (g % (BATCH // BBLK)) * BBLK
        pltpu.async_copy(
            buf, out_hbm.at[fld, pl.ds(0, EMB_DIM), pl.ds(b0, BBLK)], sem
        )

    def drain(buf, sem):
        pltpu.make_async_copy(
            out_hbm.at[0, pl.ds(0, EMB_DIM), pl.ds(0, BBLK)], buf, sem
        ).wait()

    def ring(j, carry):
        for half in range(NBUF):
            i = j * NBUF + half

            @pl.when(j > 0)
            def _():
                drain(bufs[half], sems[half])

            build_block(i, bufs[half])
            fire(i, bufs[half], sems[half])
        return carry

    lax.fori_loop(0, BPW // NBUF, ring, 0)
    for half in range(NBUF):
        drain(bufs[half], sems[half])


def kernel(x_cat, tables):
    xt_flat = x_cat.astype(jnp.int32).T.reshape(-1)
    tbl = jnp.pad(
        tables.transpose(0, 2, 1), ((0, 0), (0, 0), (0, VOCAB_PAD - VOCAB))
    ).reshape(-1)
    out = _emb_lookup(xt_flat, tbl)
    return jnp.transpose(out, (2, 0, 1))
